# Initial kernel scaffold; baseline (speedup 1.0000x reference)
import jax
import jax.numpy as jnp
from jax.experimental import pallas as pl


def kernel(feats, segment_ids, W1, b1, W2, b2):
    raise NotImplementedError("write your pallas kernel here")


# speedup vs baseline: 3.6569x; 3.6569x over previous
import functools

import jax
import jax.numpy as jnp
from jax import lax
from jax.experimental import pallas as pl
from jax.experimental.pallas import tpu as pltpu
from jax.experimental.pallas import tpu_sc as plsc

_N_ATOMS = 320000
_N_MOLS = 10000
_FEAT = 128
_HID = 64

_NC = 2
_NS = 16
_NW = _NC * _NS
_ATOMS_PER_TILE = _N_ATOMS // _NW
_CHUNK = 80
_N_CHUNKS = _ATOMS_PER_TILE // _CHUNK
_MOLS_PAD = 10240
_MOLS_PER_TILE = _MOLS_PAD // _NS

_LOG2 = 0.6931471805599453


@functools.partial(
    pl.kernel,
    mesh=plsc.VectorSubcoreMesh(core_axis_name="c", subcore_axis_name="s"),
    out_type=jax.ShapeDtypeStruct((_NC, _MOLS_PAD, _FEAT), jnp.float32),
    scratch_types=[
        pltpu.VMEM((_CHUNK,), jnp.int32),
        pltpu.VMEM((_CHUNK, _FEAT), jnp.float32),
        pltpu.VMEM_SHARED((_MOLS_PAD, _FEAT), jnp.float32),
    ],
)
def _segsum_sc(feats_hbm, ids_hbm, out_hbm, idx_v, rows_v, acc_sh):
    c = lax.axis_index("c")
    s = lax.axis_index("s")

    zeros16 = jnp.zeros((16,), jnp.float32)

    def zb(i, carry):
        rows_v[i // (_FEAT // 16), pl.ds((i % (_FEAT // 16)) * 16, 16)] = zeros16
        return carry

    lax.fori_loop(0, _CHUNK * (_FEAT // 16), zb, 0)

    mol_base = s * _MOLS_PER_TILE
    for j in range(_MOLS_PER_TILE // _CHUNK):
        pltpu.sync_copy(rows_v, acc_sh.at[pl.ds(mol_base + j * _CHUNK, _CHUNK)])
    plsc.subcore_barrier()

    tile_base = (c * _NS + s) * _ATOMS_PER_TILE

    def body(i, carry):
        a = tile_base + i * _CHUNK
        pltpu.sync_copy(ids_hbm.at[pl.ds(a, _CHUNK)], idx_v)
        pltpu.sync_copy(feats_hbm.at[pl.ds(a, _CHUNK)], rows_v)
        pltpu.sync_copy(rows_v, acc_sh.at[idx_v], add=True)
        return carry

    lax.fori_loop(0, _N_CHUNKS, body, 0)
    plsc.subcore_barrier()

    pltpu.sync_copy(acc_sh.at[pl.ds(mol_base, _MOLS_PER_TILE)],
                    out_hbm.at[c, pl.ds(mol_base, _MOLS_PER_TILE)])


_ROWS_BLK = 1000


def _mlp_body(p_ref, w1_ref, b1_ref, w2_ref, b2_ref, out_ref, fp_ref):
    fp = p_ref[0] + p_ref[1]
    fp_ref[...] = fp
    h = jnp.dot(fp, w1_ref[...], preferred_element_type=jnp.float32) + b1_ref[...]
    sp = jnp.maximum(h, 0.0) + jnp.log(1.0 + jnp.exp(-jnp.abs(h))) - _LOG2
    out_ref[...] = jnp.sum(sp * w2_ref[...], axis=1, keepdims=True) + b2_ref[...]


def _mlp_tc(partials, W1, b1r, w2r, b2r):
    grid = (_N_MOLS // _ROWS_BLK,)
    return pl.pallas_call(
        _mlp_body,
        grid=grid,
        in_specs=[
            pl.BlockSpec((_NC, _ROWS_BLK, _FEAT), lambda i: (0, i, 0)),
            pl.BlockSpec((_FEAT, _HID), lambda i: (0, 0)),
            pl.BlockSpec((1, _HID), lambda i: (0, 0)),
            pl.BlockSpec((1, _HID), lambda i: (0, 0)),
            pl.BlockSpec((1, 1), lambda i: (0, 0)),
        ],
        out_specs=[
            pl.BlockSpec((_ROWS_BLK, 1), lambda i: (i, 0)),
            pl.BlockSpec((_ROWS_BLK, _FEAT), lambda i: (i, 0)),
        ],
        out_shape=[
            jax.ShapeDtypeStruct((_N_MOLS, 1), jnp.float32),
            jax.ShapeDtypeStruct((_N_MOLS, _FEAT), jnp.float32),
        ],
    )(partials, W1, b1r, w2r, b2r)


def kernel(feats, segment_ids, W1, b1, W2, b2):
    ids = segment_ids.astype(jnp.int32)
    partials = _segsum_sc(feats, ids)
    out2d, mol_fp = _mlp_tc(partials, W1, b1.reshape(1, _HID),
                            W2.reshape(1, _HID), b2.reshape(1, 1))
    return out2d.reshape(-1), mol_fp


# speedup vs baseline: 6.9342x; 1.8962x over previous
import functools

import jax
import jax.numpy as jnp
from jax import lax
from jax.experimental import pallas as pl
from jax.experimental.pallas import tpu as pltpu
from jax.experimental.pallas import tpu_sc as plsc

_N_ATOMS = 320000
_N_MOLS = 10000
_FEAT = 128
_HID = 64

_NC = 2
_NS = 16
_NW = _NC * _NS
_ATOMS_PER_TILE = _N_ATOMS // _NW
_CHUNK = 80
_N_CHUNKS = _ATOMS_PER_TILE // _CHUNK
_MOLS_PAD = 10240
_MOLS_PER_TILE = _MOLS_PAD // _NS

_LOG2 = 0.6931471805599453


@functools.partial(
    pl.kernel,
    mesh=plsc.VectorSubcoreMesh(core_axis_name="c", subcore_axis_name="s"),
    out_type=jax.ShapeDtypeStruct((_NC, _MOLS_PAD, _FEAT), jnp.float32),
    scratch_types=[
        pltpu.VMEM((2, _CHUNK), jnp.int32),
        pltpu.VMEM((2, _CHUNK, _FEAT), jnp.float32),
        pltpu.VMEM_SHARED((_MOLS_PAD, _FEAT), jnp.float32),
        pltpu.SemaphoreType.DMA,
        pltpu.SemaphoreType.DMA,
        pltpu.SemaphoreType.DMA,
        pltpu.SemaphoreType.DMA,
    ],
)
def _segsum_sc(feats_hbm, ids_hbm, out_hbm, ibufs, bufs, acc_sh,
               fsem0, fsem1, isem0, isem1):
    c = lax.axis_index("c")
    s = lax.axis_index("s")
    w = c * _NS + s
    tile_base = w * _ATOMS_PER_TILE
    fsems = (fsem0, fsem1)
    isems = (isem0, isem1)

    zeros16 = jnp.zeros((16,), jnp.float32)

    def zb(i, carry):
        bufs[0, i // (_FEAT // 16), pl.ds((i % (_FEAT // 16)) * 16, 16)] = zeros16
        return carry

    lax.fori_loop(0, _CHUNK * (_FEAT // 16), zb, 0)

    mol_base = s * _MOLS_PER_TILE
    for j in range(_MOLS_PER_TILE // _CHUNK):
        pltpu.sync_copy(bufs.at[0], acc_sh.at[pl.ds(mol_base + j * _CHUNK, _CHUNK)])
    plsc.subcore_barrier()

    def start_load(i, b):
        a = tile_base + i * _CHUNK
        pltpu.async_copy(ids_hbm.at[pl.ds(a, _CHUNK)], ibufs.at[b], isems[b])
        pltpu.async_copy(feats_hbm.at[pl.ds(a, _CHUNK)], bufs.at[b], fsems[b])

    def wait_load(i, b):
        a = tile_base + i * _CHUNK
        pltpu.make_async_copy(ids_hbm.at[pl.ds(a, _CHUNK)], ibufs.at[b],
                              isems[b]).wait()
        pltpu.make_async_copy(feats_hbm.at[pl.ds(a, _CHUNK)], bufs.at[b],
                              fsems[b]).wait()

    def scatter(i, b):
        pltpu.sync_copy(bufs.at[b], acc_sh.at[ibufs.at[b]], add=True)

    start_load(0, 0)
    start_load(1, 1)

    def body(i0, carry):
        for b in range(2):
            i = i0 * 2 + b
            wait_load(i, b)
            scatter(i, b)

            @pl.when(i + 2 < _N_CHUNKS)
            def _():
                start_load(i + 2, b)

        return carry

    lax.fori_loop(0, (_N_CHUNKS - 1) // 2, body, 0)
    wait_load(_N_CHUNKS - 1, 0)
    scatter(_N_CHUNKS - 1, 0)
    plsc.subcore_barrier()

    pltpu.sync_copy(acc_sh.at[pl.ds(mol_base, _MOLS_PER_TILE)],
                    out_hbm.at[c, pl.ds(mol_base, _MOLS_PER_TILE)])


_ROWS_BLK = 1000


def _mlp_body(p_ref, w1_ref, b1_ref, w2_ref, b2_ref, out_ref, fp_ref):
    fp = p_ref[0] + p_ref[1]
    fp_ref[...] = fp
    h = jnp.dot(fp, w1_ref[...], preferred_element_type=jnp.float32) + b1_ref[...]
    sp = jnp.maximum(h, 0.0) + jnp.log(1.0 + jnp.exp(-jnp.abs(h))) - _LOG2
    out_ref[...] = jnp.sum(sp * w2_ref[...], axis=1, keepdims=True) + b2_ref[...]


def _mlp_tc(partials, W1, b1r, w2r, b2r):
    grid = (_N_MOLS // _ROWS_BLK,)
    return pl.pallas_call(
        _mlp_body,
        grid=grid,
        in_specs=[
            pl.BlockSpec((_NC, _ROWS_BLK, _FEAT), lambda i: (0, i, 0)),
            pl.BlockSpec((_FEAT, _HID), lambda i: (0, 0)),
            pl.BlockSpec((1, _HID), lambda i: (0, 0)),
            pl.BlockSpec((1, _HID), lambda i: (0, 0)),
            pl.BlockSpec((1, 1), lambda i: (0, 0)),
        ],
        out_specs=[
            pl.BlockSpec((_ROWS_BLK, 1), lambda i: (i, 0)),
            pl.BlockSpec((_ROWS_BLK, _FEAT), lambda i: (i, 0)),
        ],
        out_shape=[
            jax.ShapeDtypeStruct((_N_MOLS, 1), jnp.float32),
            jax.ShapeDtypeStruct((_N_MOLS, _FEAT), jnp.float32),
        ],
    )(partials, W1, b1r, w2r, b2r)


def kernel(feats, segment_ids, W1, b1, W2, b2):
    ids = segment_ids.astype(jnp.int32)
    partials = _segsum_sc(feats, ids)
    out2d, mol_fp = _mlp_tc(partials, W1, b1.reshape(1, _HID),
                            W2.reshape(1, _HID), b2.reshape(1, 1))
    return out2d.reshape(-1), mol_fp


# speedup vs baseline: 7.1123x; 1.0257x over previous
import functools

import jax
import jax.numpy as jnp
from jax import lax
from jax.experimental import pallas as pl
from jax.experimental.pallas import tpu as pltpu
from jax.experimental.pallas import tpu_sc as plsc

_N_ATOMS = 320000
_N_MOLS = 10000
_FEAT = 128
_HID = 64

_NC = 2
_NS = 16
_NW = _NC * _NS
_ATOMS_PER_TILE = _N_ATOMS // _NW
_CHUNK = 80
_N_CHUNKS = _ATOMS_PER_TILE // _CHUNK
_MOLS_PAD = 10240
_MOLS_PER_TILE = _MOLS_PAD // _NS

_LOG2 = 0.6931471805599453


@functools.partial(
    pl.kernel,
    mesh=plsc.VectorSubcoreMesh(core_axis_name="c", subcore_axis_name="s"),
    out_type=jax.ShapeDtypeStruct((_NC, _MOLS_PAD, _FEAT), jnp.float32),
    scratch_types=[
        pltpu.VMEM((4, _CHUNK), jnp.int32),
        pltpu.VMEM((4, _CHUNK, _FEAT), jnp.float32),
        pltpu.VMEM_SHARED((_MOLS_PAD, _FEAT), jnp.float32),
        pltpu.SemaphoreType.DMA,
        pltpu.SemaphoreType.DMA,
        pltpu.SemaphoreType.DMA,
        pltpu.SemaphoreType.DMA,
        pltpu.SemaphoreType.DMA,
        pltpu.SemaphoreType.DMA,
        pltpu.SemaphoreType.DMA,
        pltpu.SemaphoreType.DMA,
        pltpu.SemaphoreType.DMA,
        pltpu.SemaphoreType.DMA,
        pltpu.SemaphoreType.DMA,
        pltpu.SemaphoreType.DMA,
    ],
)
def _segsum_sc(feats_hbm, ids_hbm, out_hbm, ibufs, bufs, acc_sh,
               fsem0, fsem1, fsem2, fsem3, isem0, isem1, isem2, isem3,
               ssem0, ssem1, ssem2, ssem3):
    c = lax.axis_index("c")
    s = lax.axis_index("s")
    w = c * _NS + s
    tile_base = w * _ATOMS_PER_TILE
    fsems = (fsem0, fsem1, fsem2, fsem3)
    isems = (isem0, isem1, isem2, isem3)
    ssems = (ssem0, ssem1, ssem2, ssem3)

    zeros16 = jnp.zeros((16,), jnp.float32)

    def zb(i, carry):
        bufs[0, i // (_FEAT // 16), pl.ds((i % (_FEAT // 16)) * 16, 16)] = zeros16
        return carry

    lax.fori_loop(0, _CHUNK * (_FEAT // 16), zb, 0)

    mol_base = s * _MOLS_PER_TILE
    for j in range(_MOLS_PER_TILE // _CHUNK):
        pltpu.sync_copy(bufs.at[0], acc_sh.at[pl.ds(mol_base + j * _CHUNK, _CHUNK)])
    plsc.subcore_barrier()

    def start_load(i, b):
        a = tile_base + i * _CHUNK
        pltpu.async_copy(ids_hbm.at[pl.ds(a, _CHUNK)], ibufs.at[b], isems[b])
        pltpu.async_copy(feats_hbm.at[pl.ds(a, _CHUNK)], bufs.at[b], fsems[b])

    def wait_load(i, b):
        a = tile_base + i * _CHUNK
        pltpu.make_async_copy(ids_hbm.at[pl.ds(a, _CHUNK)], ibufs.at[b],
                              isems[b]).wait()
        pltpu.make_async_copy(feats_hbm.at[pl.ds(a, _CHUNK)], bufs.at[b],
                              fsems[b]).wait()

    def start_scatter(b):
        pltpu.async_copy(bufs.at[b], acc_sh.at[ibufs.at[b]], ssems[b], add=True)

    def wait_scatter(b):
        pltpu.make_async_copy(bufs.at[b], acc_sh.at[ibufs.at[b]],
                              ssems[b]).wait()

    for l in range(4):
        start_load(l, l)

    def body(j0, carry):
        for b in range(4):
            j = j0 * 4 + b
            wait_load(j, b)

            @pl.when(j >= 2)
            def _():
                wait_scatter((b - 2) % 4)

            start_scatter(b)

            @pl.when((j >= 2) & (j + 2 < _N_CHUNKS))
            def _():
                start_load(j + 2, (b + 2) % 4)

        return carry

    lax.fori_loop(0, _N_CHUNKS // 4, body, 0)
    wait_load(_N_CHUNKS - 1, 0)
    wait_scatter(2)
    start_scatter(0)
    wait_scatter(3)
    wait_scatter(0)
    plsc.subcore_barrier()

    pltpu.sync_copy(acc_sh.at[pl.ds(mol_base, _MOLS_PER_TILE)],
                    out_hbm.at[c, pl.ds(mol_base, _MOLS_PER_TILE)])


_ROWS_BLK = 1000


def _mlp_body(p_ref, w1_ref, b1_ref, w2_ref, b2_ref, out_ref, fp_ref):
    fp = p_ref[0] + p_ref[1]
    fp_ref[...] = fp
    h = jnp.dot(fp, w1_ref[...], preferred_element_type=jnp.float32) + b1_ref[...]
    sp = jnp.maximum(h, 0.0) + jnp.log(1.0 + jnp.exp(-jnp.abs(h))) - _LOG2
    out_ref[...] = jnp.sum(sp * w2_ref[...], axis=1, keepdims=True) + b2_ref[...]


def _mlp_tc(partials, W1, b1r, w2r, b2r):
    grid = (_N_MOLS // _ROWS_BLK,)
    return pl.pallas_call(
        _mlp_body,
        grid=grid,
        in_specs=[
            pl.BlockSpec((_NC, _ROWS_BLK, _FEAT), lambda i: (0, i, 0)),
            pl.BlockSpec((_FEAT, _HID), lambda i: (0, 0)),
            pl.BlockSpec((1, _HID), lambda i: (0, 0)),
            pl.BlockSpec((1, _HID), lambda i: (0, 0)),
            pl.BlockSpec((1, 1), lambda i: (0, 0)),
        ],
        out_specs=[
            pl.BlockSpec((_ROWS_BLK, 1), lambda i: (i, 0)),
            pl.BlockSpec((_ROWS_BLK, _FEAT), lambda i: (i, 0)),
        ],
        out_shape=[
            jax.ShapeDtypeStruct((_N_MOLS, 1), jnp.float32),
            jax.ShapeDtypeStruct((_N_MOLS, _FEAT), jnp.float32),
        ],
    )(partials, W1, b1r, w2r, b2r)


def kernel(feats, segment_ids, W1, b1, W2, b2):
    ids = segment_ids.astype(jnp.int32)
    partials = _segsum_sc(feats, ids)
    out2d, mol_fp = _mlp_tc(partials, W1, b1.reshape(1, _HID),
                            W2.reshape(1, _HID), b2.reshape(1, 1))
    return out2d.reshape(-1), mol_fp
